# Initial kernel scaffold; baseline (speedup 1.0000x reference)
import jax
import jax.numpy as jnp
from jax.experimental import pallas as pl


def kernel(edge_index, attn, review_feat, cj, ci, weight, prob_score_w, review_score_w, review_w):
    raise NotImplementedError("write your pallas kernel here")


# speedup vs baseline: 3.5020x; 3.5020x over previous
import dataclasses
import functools

import jax
import jax.numpy as jnp
from jax import lax
from jax.experimental import pallas as pl
from jax.experimental.pallas import tpu as pltpu
from jax.experimental.pallas import tpu_sc as plsc

N = 10000
E = 320000
D = 128

NC = 2
NS = 16
NW = NC * NS
EPW = E // NW
BLK = 80
NBLK = EPW // BLK
NPAD = 10240
RPT = NPAD // NS
ZROWS = 128

def _sc_compiler_params():
    cp = pltpu.CompilerParams()
    if "needs_layout_passes" in pltpu.CompilerParams.__dataclass_fields__:
        cp = dataclasses.replace(cp, needs_layout_passes=False)
    return cp


def _sc_gather_body(src_hbm, w_hbm, cj_hbm, g1_hbm, g2_hbm,
                    idx_v, rows_v, cj_v, g2buf_v, sem1):
    wid = lax.axis_index("s") * NC + lax.axis_index("c")
    base = wid * EPW
    pltpu.sync_copy(cj_hbm, cj_v)

    @pl.loop(0, NBLK)
    def _(b):
        off = base + b * BLK
        pltpu.sync_copy(src_hbm.at[pl.ds(off, BLK)], idx_v)
        c1 = pltpu.async_copy(w_hbm.at[idx_v], rows_v, sem1)
        @pl.loop(0, BLK // 16)
        def _(k):
            iv = idx_v[pl.ds(k * 16, 16)]
            g2buf_v[pl.ds(k * 16, 16)] = plsc.load_gather(cj_v, [iv])
        c1.wait()
        pltpu.sync_copy(rows_v, g1_hbm.at[pl.ds(off, BLK)])
        pltpu.sync_copy(g2buf_v, g2_hbm.at[pl.ds(off, BLK)])


@functools.cache
def _build_sc_gather():
    mesh = plsc.VectorSubcoreMesh(
        core_axis_name="c", subcore_axis_name="s",
        num_cores=NC, num_subcores=NS)
    return pl.kernel(
        _sc_gather_body,
        out_type=[jax.ShapeDtypeStruct((E, D), jnp.float32),
                  jax.ShapeDtypeStruct((E,), jnp.float32)],
        mesh=mesh,
        scratch_types=[pltpu.VMEM((BLK,), jnp.int32),
                       pltpu.VMEM((BLK, D), jnp.float32),
                       pltpu.VMEM((N,), jnp.float32),
                       pltpu.VMEM((BLK,), jnp.float32),
                       pltpu.SemaphoreType.DMA],
        compiler_params=_sc_compiler_params(),
    )


BE = 512
NBE = E // BE


def _tc_main_body(x_ref, attn_ref, g1_ref, g2_ref, wT_ref, pw_ref, rw_ref, m_ref):
    x = x_ref[...]
    rf = jnp.dot(x, wT_ref[...], preferred_element_type=jnp.float32)
    pa_lin = jnp.sum(x * pw_ref[...], axis=1, keepdims=True)
    ra_lin = jnp.sum(x * rw_ref[...], axis=1, keepdims=True)
    pa = 1.0 / (1.0 + jnp.exp(-pa_lin))
    ra = 1.0 / (1.0 + jnp.exp(-ra_lin))
    cj_src = g2_ref[...]
    m_ref[...] = (g1_ref[...] * pa + rf * (ra * attn_ref[...])) * cj_src


_tc_main = pl.pallas_call(
    _tc_main_body,
    grid=(NBE,),
    in_specs=[
        pl.BlockSpec((BE, D), lambda i: (i, 0)),
        pl.BlockSpec((BE, 1), lambda i: (i, 0)),
        pl.BlockSpec((BE, D), lambda i: (i, 0)),
        pl.BlockSpec((BE, 1), lambda i: (i, 0)),
        pl.BlockSpec((D, D), lambda i: (0, 0)),
        pl.BlockSpec((1, D), lambda i: (0, 0)),
        pl.BlockSpec((1, D), lambda i: (0, 0)),
    ],
    out_specs=pl.BlockSpec((BE, D), lambda i: (i, 0)),
    out_shape=jax.ShapeDtypeStruct((E, D), jnp.float32),
)


def _sc_scatter_body(dst_hbm, m_hbm, z_hbm, out_hbm, idx_v, rows_v, acc_sh, sem):
    cid = lax.axis_index("c")
    sid = lax.axis_index("s")
    wid = sid * NC + cid
    base = wid * EPW

    @pl.loop(0, RPT // ZROWS)
    def _(k):
        pltpu.sync_copy(z_hbm, acc_sh.at[pl.ds(sid * RPT + k * ZROWS, ZROWS)])
    plsc.subcore_barrier()

    @pl.loop(0, NBLK)
    def _(b):
        off = base + b * BLK
        pltpu.sync_copy(dst_hbm.at[pl.ds(off, BLK)], idx_v)
        pltpu.sync_copy(m_hbm.at[pl.ds(off, BLK)], rows_v)
        pltpu.sync_copy(rows_v, acc_sh.at[idx_v], add=True)

    plsc.subcore_barrier()

    @pl.loop(0, RPT // ZROWS)
    def _(k):
        r = sid * RPT + k * ZROWS
        pltpu.sync_copy(acc_sh.at[pl.ds(r, ZROWS)], out_hbm.at[cid].at[pl.ds(r, ZROWS)])


@functools.cache
def _build_sc_scatter():
    mesh = plsc.VectorSubcoreMesh(
        core_axis_name="c", subcore_axis_name="s",
        num_cores=NC, num_subcores=NS)
    return pl.kernel(
        _sc_scatter_body,
        out_type=jax.ShapeDtypeStruct((NC, NPAD, D), jnp.float32),
        mesh=mesh,
        scratch_types=[pltpu.VMEM((BLK,), jnp.int32),
                       pltpu.VMEM((BLK, D), jnp.float32),
                       pltpu.VMEM_SHARED((NPAD, D), jnp.float32),
                       pltpu.SemaphoreType.DMA],
        compiler_params=_sc_compiler_params(),
    )


BN = 1000
NBN = N // BN


def _tc_combine_body(p_ref, ci_ref, o_ref):
    o_ref[...] = (p_ref[0] + p_ref[1]) * ci_ref[...]


_tc_combine = pl.pallas_call(
    _tc_combine_body,
    grid=(NBN,),
    in_specs=[
        pl.BlockSpec((NC, BN, D), lambda i: (0, i, 0)),
        pl.BlockSpec((BN, 1), lambda i: (i, 0)),
    ],
    out_specs=pl.BlockSpec((BN, D), lambda i: (i, 0)),
    out_shape=jax.ShapeDtypeStruct((N, D), jnp.float32),
)


def kernel(edge_index, attn, review_feat, cj, ci, weight, prob_score_w,
           review_score_w, review_w):
    src = edge_index[0]
    dst = edge_index[1]
    attn2 = attn.reshape(E, 1)
    zeros = jnp.zeros((ZROWS, D), jnp.float32)

    g1, g2 = _build_sc_gather()(src, weight, cj.reshape(N))
    m = _tc_main(review_feat, attn2, g1, g2.reshape(E, 1), review_w.T,
                 prob_score_w, review_score_w)
    parts = _build_sc_scatter()(dst, m, zeros)
    return _tc_combine(parts, ci)
